# baseline (speedup 1.0000x reference)
import dataclasses
import functools

import jax
import jax.numpy as jnp
from jax import lax
from jax.experimental import pallas as pl
from jax.experimental.pallas import tpu as pltpu
from jax.experimental.pallas import tpu_sc as plsc

N = 10000
E = 160000
NA = 10240
RPT = NA // 16
WE = 64
WPT = 160
EPT = WPT * WE
E_SC = 16 * EPT


def _dense_kernel(nchunks, x_ref, w_ref, ast_ref, *out_refs):
    x = x_ref[...]
    w = w_ref[0]
    h = jax.lax.dot_general(x, w, (((1,), (0,)), ((), ())),
                            preferred_element_type=jnp.float32)
    for k in range(nchunks):
        out_refs[k][0] = h[:, k * 128:(k + 1) * 128]
    out_refs[nchunks][0] = jax.lax.dot_general(
        h, ast_ref[0], (((1,), (1,)), ((), ())),
        preferred_element_type=jnp.float32)


def _dense_heads(x, W, a, bn=400):
    H, Din, u = W.shape
    nchunks = u // 128
    nb = N // bn
    a2 = a[:, :, 0].reshape(H, 2, u)
    ast = jnp.concatenate([a2, jnp.zeros((H, 6, u), jnp.float32)], axis=1)
    outs = pl.pallas_call(
        functools.partial(_dense_kernel, nchunks),
        grid=(H, nb),
        in_specs=[
            pl.BlockSpec((bn, Din), lambda i, n: (n, 0)),
            pl.BlockSpec((1, Din, u), lambda i, n: (i, 0, 0)),
            pl.BlockSpec((1, 8, u), lambda i, n: (i, 0, 0)),
        ],
        out_specs=[pl.BlockSpec((1, bn, 128), lambda i, n: (i, n, 0))
                   for _ in range(nchunks)]
                  + [pl.BlockSpec((1, bn, 8), lambda i, n: (i, n, 0))],
        out_shape=[jax.ShapeDtypeStruct((H, N, 128), jnp.float32)
                   for _ in range(nchunks)]
                  + [jax.ShapeDtypeStruct((H, N, 8), jnp.float32)],
    )(x, W, ast)
    h_chunks, scal = outs[:nchunks], outs[nchunks]
    return h_chunks, scal[:, :, 0], scal[:, :, 1]


def _sc_gat_body(relu, pair_base, nchunks, *refs):
    h_hbms = refs[:nchunks]
    sdp_hbm, ss_hbm, ds_hbm, out_hbm = refs[nchunks:nchunks + 4]
    (sdp_loc, ss_loc, rows0, rows1, dsb0, dsb1, dstw0, dstw1,
     gidx0, gidx1, sw0, sw1, dloc, zvec,
     g_sem0, g_sem1, t_sem0, t_sem1,
     acc_sh, den_sh) = refs[nchunks + 4:]
    rows = (rows0, rows1)
    dsb = (dsb0, dsb1)
    dstw = (dstw0, dstw1)
    gidx = (gidx0, gidx1)
    sw = (sw0, sw1)
    g_sem = (g_sem0, g_sem1)
    t_sem = (t_sem0, t_sem1)

    c = lax.axis_index("c")
    s = lax.axis_index("s")
    head = pair_base + c
    zero16 = jnp.zeros((16,), jnp.float32)

    pltpu.sync_copy(sdp_hbm.at[pl.ds(head * NA, NA)], sdp_loc)
    pltpu.sync_copy(ss_hbm.at[pl.ds(head * N, N)], ss_loc)

    def zero_rows(r_ref):
        @pl.loop(0, WE)
        def _z(r):
            for j in range(8):
                r_ref[r, pl.ds(j * 16, 16)] = zero16

    def zero_acc():
        for k in range(RPT // WE):
            pltpu.sync_copy(rows0, acc_sh.at[pl.ds(s * RPT + k * WE, WE)])

    zero_rows(rows0)

    @pl.loop(0, RPT // 16)
    def _zv(i):
        zvec[pl.ds(i * 16, 16)] = zero16

    zero_acc()
    pltpu.sync_copy(zvec, den_sh.at[pl.ds(s * RPT, RPT)])
    plsc.subcore_barrier()

    def load_idx(w, b):
        return pltpu.make_async_copy(
            ds_hbm.at[pl.ds((s * WPT + w) * 2 * WE, 2 * WE)], dsb[b], g_sem[b])

    def scores(w, b):
        for j in range(WE // 16):
            di = dsb[b][pl.ds(j * 16, 16)]
            si = dsb[b][pl.ds(WE + j * 16, 16)]
            t = plsc.load_gather(sdp_loc, [di]) + plsc.load_gather(ss_loc, [si])
            t = jnp.where(t > 0, t, 0.2 * t)
            t = jnp.minimum(jnp.maximum(t, -2.0), 2.0)
            sw[b][pl.ds(j * 16, 16)] = jnp.exp(t)
            gidx[b][pl.ds(j * 16, 16)] = si + head * N
            dstw[b][pl.ds(j * 16, 16)] = di

    def gather(b, h_hbm):
        return pltpu.make_async_copy(h_hbm.at[gidx[b]], rows[b], g_sem[b])

    def scatter(b):
        return pltpu.make_async_copy(rows[b], acc_sh.at[dstw[b]], t_sem[b])

    def scale(b):
        @pl.loop(0, WE)
        def _scale(r):
            sv = plsc.load_gather(sw[b], [jnp.full((16,), r, jnp.int32)])
            for j in range(8):
                rows[b][r, pl.ds(j * 16, 16)] = (
                    rows[b][r, pl.ds(j * 16, 16)] * sv)

    def flush(chunk):
        for k in range(RPT // WE):
            row0 = s * RPT + k * WE
            pltpu.sync_copy(acc_sh.at[pl.ds(row0, WE)], rows0)
            pltpu.sync_copy(den_sh.at[pl.ds(row0, WE)], dloc)

            @pl.loop(0, WE)
            def _norm(r):
                dv = plsc.load_gather(dloc, [jnp.full((16,), r, jnp.int32)])
                sc = 1.0 / (dv + 1e-9)
                for j in range(8):
                    v = rows0[r, pl.ds(j * 16, 16)] * sc
                    if relu:
                        v = jnp.maximum(v, 0.0)
                    rows0[r, pl.ds(j * 16, 16)] = v

            pltpu.sync_copy(
                rows0, out_hbm.at[pl.ds((c * nchunks + chunk) * NA + row0,
                                        WE)])

    def edge_pass_sync(chunk, h_hbm):
        first = chunk == 0

        @pl.loop(0, WPT)
        def _win(w):
            pltpu.sync_copy(
                ds_hbm.at[pl.ds((s * WPT + w) * 2 * WE, 2 * WE)], dsb0)
            scores(w, 0)
            pltpu.sync_copy(h_hbm.at[gidx0], rows0)
            if first:
                pltpu.sync_copy(sw0, den_sh.at[dstw0], add=True)
            scale(0)
            pltpu.sync_copy(rows0, acc_sh.at[dstw0], add=True)

        plsc.subcore_barrier()
        flush(chunk)
        if chunk + 1 < nchunks:
            plsc.subcore_barrier()
            zero_rows(rows0)
            zero_acc()
            plsc.subcore_barrier()

    def edge_pass(chunk, h_hbm):
        first = chunk == 0

        def body(w, b, drain=True, prefetch=True, more_idx=True):
            nb = 1 - b
            if prefetch:
                if drain:
                    scatter(nb).wait()
                load_idx(w + 1, nb).wait()
                scores(w + 1, nb)
                gather(nb, h_hbm).start()
            gather(b, h_hbm).wait()
            if more_idx:
                load_idx(w + 2, b).start()
            if first:
                pltpu.sync_copy(sw[b], den_sh.at[dstw[b]], add=True)
            scale(b)
            scatter(b).start()

        load_idx(0, 0).start()
        load_idx(0, 0).wait()
        scores(0, 0)
        gather(0, h_hbm).start()
        load_idx(1, 1).start()
        body(0, 0, drain=False)

        @pl.loop(1, WPT - 3, step=2)
        def _pair(w0):
            body(w0, 1)
            body(w0 + 1, 0)

        body(WPT - 3, 1)
        body(WPT - 2, 0, more_idx=False)
        body(WPT - 1, 1, prefetch=False, more_idx=False)
        scatter(0).wait()
        scatter(1).wait()
        plsc.subcore_barrier()
        flush(chunk)
        if chunk + 1 < nchunks:
            plsc.subcore_barrier()
            zero_rows(rows0)
            zero_acc()
            plsc.subcore_barrier()

    for chunk in range(nchunks):
        edge_pass_sync(chunk, h_hbms[chunk])


def _sc_compiler_params():
    cp = pltpu.CompilerParams()
    if "needs_layout_passes" in pltpu.CompilerParams.__dataclass_fields__:
        cp = dataclasses.replace(cp, needs_layout_passes=False)
    return cp


def _sc_pair(h_chunks, sdp_flat, ss_flat, ds_packed, pair_base, relu):
    nchunks = len(h_chunks)
    mesh = plsc.VectorSubcoreMesh(core_axis_name="c", subcore_axis_name="s")
    body = functools.partial(_sc_gat_body, relu, pair_base, nchunks)
    run = pl.kernel(
        body,
        out_type=jax.ShapeDtypeStruct((2 * nchunks * NA, 128), jnp.float32),
        mesh=mesh,
        scratch_types=[
            pltpu.VMEM((NA,), jnp.float32),
            pltpu.VMEM((N,), jnp.float32),
            pltpu.VMEM((WE, 128), jnp.float32),
            pltpu.VMEM((WE, 128), jnp.float32),
            pltpu.VMEM((2 * WE,), jnp.int32),
            pltpu.VMEM((2 * WE,), jnp.int32),
            pltpu.VMEM((WE,), jnp.int32),
            pltpu.VMEM((WE,), jnp.int32),
            pltpu.VMEM((WE,), jnp.int32),
            pltpu.VMEM((WE,), jnp.int32),
            pltpu.VMEM((WE,), jnp.float32),
            pltpu.VMEM((WE,), jnp.float32),
            pltpu.VMEM((WE,), jnp.float32),
            pltpu.VMEM((RPT,), jnp.float32),
            pltpu.SemaphoreType.DMA,
            pltpu.SemaphoreType.DMA,
            pltpu.SemaphoreType.DMA,
            pltpu.SemaphoreType.DMA,
            pltpu.VMEM_SHARED((NA, 128), jnp.float32),
            pltpu.VMEM_SHARED((NA,), jnp.float32),
        ],
        compiler_params=_sc_compiler_params(),
    )
    return run(*h_chunks, sdp_flat, ss_flat, ds_packed)


def _gat_layer(x, ds_packed, W, a, merge):
    H = W.shape[0]
    h_chunks, sd, ss = _dense_heads(x, W, a)
    nchunks = len(h_chunks)
    sdp = jnp.pad(sd, ((0, 0), (0, NA - N))).reshape(-1)
    ssf = ss.reshape(-1)
    flat = [hh.reshape(H * N, 128) for hh in h_chunks]
    outs = {}
    for p in range(H // 2):
        o = _sc_pair(flat, sdp, ssf, ds_packed, 2 * p, merge == "concat")
        o = o.reshape(2, nchunks, NA, 128)
        for cc in range(2):
            for ci in range(nchunks):
                outs[(2 * p + cc, ci)] = o[cc, ci, :N]
    if merge == "concat":
        cols = [outs[(hd, ci)] for hd in range(H) for ci in range(nchunks)]
        return jnp.concatenate(cols, axis=1)
    acc = outs[(0, 0)]
    for hd in range(1, H):
        acc = acc + outs[(hd, 0)]
    return jax.nn.relu(acc / H)


def kernel(node_states, edges, training, W1, a1, W2, a2, W3, a3):
    dst = edges[:, 0]
    src = edges[:, 1]
    pad = E_SC - E
    e = jnp.arange(pad, dtype=jnp.int32)
    dst_sc = jnp.concatenate([dst, N + (e % 64)])
    src_sc = jnp.concatenate([src, e % N])
    ds_packed = jnp.concatenate(
        [dst_sc.reshape(-1, WE), src_sc.reshape(-1, WE)], axis=1).reshape(-1)
    x = _gat_layer(node_states, ds_packed, W1, a1, "concat")
    x = _gat_layer(x, ds_packed, W2, a2, "concat")
    return _gat_layer(x, ds_packed, W3, a3, "avg")


# speedup vs baseline: 1.4525x; 1.4525x over previous
import dataclasses
import functools

import jax
import jax.numpy as jnp
from jax import lax
from jax.experimental import pallas as pl
from jax.experimental.pallas import tpu as pltpu
from jax.experimental.pallas import tpu_sc as plsc

N = 10000
E = 160000
NA = 10240
RPT = NA // 16
WE = 64
WPT = 160
EPT = WPT * WE
E_SC = 16 * EPT


def _dense_kernel(nchunks, x_ref, w_ref, ast_ref, *out_refs):
    x = x_ref[...]
    w = w_ref[0]
    h = jax.lax.dot_general(x, w, (((1,), (0,)), ((), ())),
                            preferred_element_type=jnp.float32)
    for k in range(nchunks):
        out_refs[k][0] = h[:, k * 128:(k + 1) * 128]
    out_refs[nchunks][0] = jax.lax.dot_general(
        h, ast_ref[0], (((1,), (1,)), ((), ())),
        preferred_element_type=jnp.float32)


def _dense_heads(x, W, a, bn=400):
    H, Din, u = W.shape
    nchunks = u // 128
    nb = N // bn
    a2 = a[:, :, 0].reshape(H, 2, u)
    ast = jnp.concatenate([a2, jnp.zeros((H, 6, u), jnp.float32)], axis=1)
    outs = pl.pallas_call(
        functools.partial(_dense_kernel, nchunks),
        grid=(H, nb),
        in_specs=[
            pl.BlockSpec((bn, Din), lambda i, n: (n, 0)),
            pl.BlockSpec((1, Din, u), lambda i, n: (i, 0, 0)),
            pl.BlockSpec((1, 8, u), lambda i, n: (i, 0, 0)),
        ],
        out_specs=[pl.BlockSpec((1, bn, 128), lambda i, n: (i, n, 0))
                   for _ in range(nchunks)]
                  + [pl.BlockSpec((1, bn, 8), lambda i, n: (i, n, 0))],
        out_shape=[jax.ShapeDtypeStruct((H, N, 128), jnp.float32)
                   for _ in range(nchunks)]
                  + [jax.ShapeDtypeStruct((H, N, 8), jnp.float32)],
    )(x, W, ast)
    h_chunks, scal = outs[:nchunks], outs[nchunks]
    return h_chunks, scal[:, :, 0], scal[:, :, 1]


def _sc_gat_body(relu, pair_base, nchunks, *refs):
    h_hbms = refs[:nchunks]
    sdp_hbm, ss_hbm, ds_hbm, out_hbm = refs[nchunks:nchunks + 4]
    (sdp_loc, ss_loc, rows0, rows1, dsb0, dsb1, dstw0, dstw1,
     gidx0, gidx1, sw0, sw1, dloc, zvec,
     g_sem0, g_sem1, t_sem0, t_sem1,
     acc_sh, den_sh) = refs[nchunks + 4:]
    rows = (rows0, rows1)
    dsb = (dsb0, dsb1)
    dstw = (dstw0, dstw1)
    gidx = (gidx0, gidx1)
    sw = (sw0, sw1)
    g_sem = (g_sem0, g_sem1)
    t_sem = (t_sem0, t_sem1)

    c = lax.axis_index("c")
    s = lax.axis_index("s")
    head = pair_base + c
    zero16 = jnp.zeros((16,), jnp.float32)

    pltpu.sync_copy(sdp_hbm.at[pl.ds(head * NA, NA)], sdp_loc)
    pltpu.sync_copy(ss_hbm.at[pl.ds(head * N, N)], ss_loc)

    def zero_rows(r_ref):
        @pl.loop(0, WE)
        def _z(r):
            for j in range(8):
                r_ref[r, pl.ds(j * 16, 16)] = zero16

    def zero_acc():
        for k in range(RPT // WE):
            pltpu.sync_copy(rows0, acc_sh.at[pl.ds(s * RPT + k * WE, WE)])

    zero_rows(rows0)

    @pl.loop(0, RPT // 16)
    def _zv(i):
        zvec[pl.ds(i * 16, 16)] = zero16

    zero_acc()
    pltpu.sync_copy(zvec, den_sh.at[pl.ds(s * RPT, RPT)])
    plsc.subcore_barrier()

    def load_idx(w, b):
        pltpu.sync_copy(
            ds_hbm.at[pl.ds((s * WPT + w) * 2 * WE, 2 * WE)], dsb[b])

    def scores(w, b):
        for j in range(WE // 16):
            di = dsb[b][pl.ds(j * 16, 16)]
            si = dsb[b][pl.ds(WE + j * 16, 16)]
            t = plsc.load_gather(sdp_loc, [di]) + plsc.load_gather(ss_loc, [si])
            t = jnp.where(t > 0, t, 0.2 * t)
            t = jnp.minimum(jnp.maximum(t, -2.0), 2.0)
            sw[b][pl.ds(j * 16, 16)] = jnp.exp(t)
            gidx[b][pl.ds(j * 16, 16)] = si + head * N
            dstw[b][pl.ds(j * 16, 16)] = di

    def gather(b, h_hbm):
        return pltpu.make_async_copy(h_hbm.at[gidx[b]], rows[b], g_sem[b])

    def scatter(b):
        return pltpu.make_async_copy(rows[b], acc_sh.at[dstw[b]], t_sem[b])

    def scale(b):
        @pl.loop(0, WE)
        def _scale(r):
            sv = plsc.load_gather(sw[b], [jnp.full((16,), r, jnp.int32)])
            for j in range(8):
                rows[b][r, pl.ds(j * 16, 16)] = (
                    rows[b][r, pl.ds(j * 16, 16)] * sv)

    def flush(chunk):
        for k in range(RPT // WE):
            row0 = s * RPT + k * WE
            pltpu.sync_copy(acc_sh.at[pl.ds(row0, WE)], rows0)
            pltpu.sync_copy(den_sh.at[pl.ds(row0, WE)], dloc)

            @pl.loop(0, WE)
            def _norm(r):
                dv = plsc.load_gather(dloc, [jnp.full((16,), r, jnp.int32)])
                sc = 1.0 / (dv + 1e-9)
                for j in range(8):
                    v = rows0[r, pl.ds(j * 16, 16)] * sc
                    if relu:
                        v = jnp.maximum(v, 0.0)
                    rows0[r, pl.ds(j * 16, 16)] = v

            pltpu.sync_copy(
                rows0, out_hbm.at[pl.ds((c * nchunks + chunk) * NA + row0,
                                        WE)])

    def edge_pass_sync(chunk, h_hbm):
        first = chunk == 0

        @pl.loop(0, WPT)
        def _win(w):
            pltpu.sync_copy(
                ds_hbm.at[pl.ds((s * WPT + w) * 2 * WE, 2 * WE)], dsb0)
            scores(w, 0)
            pltpu.sync_copy(h_hbm.at[gidx0], rows0)
            if first:
                pltpu.sync_copy(sw0, den_sh.at[dstw0], add=True)
            scale(0)
            pltpu.sync_copy(rows0, acc_sh.at[dstw0], add=True)

        plsc.subcore_barrier()
        flush(chunk)
        if chunk + 1 < nchunks:
            plsc.subcore_barrier()
            zero_rows(rows0)
            zero_acc()
            plsc.subcore_barrier()

    def edge_pass(chunk, h_hbm):
        first = chunk == 0

        def body(w, b, prefetch=True):
            nb = 1 - b
            if prefetch:
                load_idx(w + 1, nb)
                scores(w + 1, nb)
                gather(nb, h_hbm).start()
            gather(b, h_hbm).wait()
            if first:
                pltpu.sync_copy(sw[b], den_sh.at[dstw[b]], add=True)
            scale(b)
            pltpu.sync_copy(rows[b], acc_sh.at[dstw[b]], add=True)

        load_idx(0, 0)
        scores(0, 0)
        gather(0, h_hbm).start()
        body(0, 0)

        @pl.loop(1, WPT - 3, step=2)
        def _pair(w0):
            body(w0, 1)
            body(w0 + 1, 0)

        body(WPT - 3, 1)
        body(WPT - 2, 0)
        body(WPT - 1, 1, prefetch=False)
        plsc.subcore_barrier()
        flush(chunk)
        if chunk + 1 < nchunks:
            plsc.subcore_barrier()
            zero_rows(rows0)
            zero_acc()
            plsc.subcore_barrier()

    for chunk in range(nchunks):
        edge_pass(chunk, h_hbms[chunk])


def _sc_compiler_params():
    cp = pltpu.CompilerParams()
    if "needs_layout_passes" in pltpu.CompilerParams.__dataclass_fields__:
        cp = dataclasses.replace(cp, needs_layout_passes=False)
    return cp


def _sc_pair(h_chunks, sdp_flat, ss_flat, ds_packed, pair_base, relu):
    nchunks = len(h_chunks)
    mesh = plsc.VectorSubcoreMesh(core_axis_name="c", subcore_axis_name="s")
    body = functools.partial(_sc_gat_body, relu, pair_base, nchunks)
    run = pl.kernel(
        body,
        out_type=jax.ShapeDtypeStruct((2 * nchunks * NA, 128), jnp.float32),
        mesh=mesh,
        scratch_types=[
            pltpu.VMEM((NA,), jnp.float32),
            pltpu.VMEM((N,), jnp.float32),
            pltpu.VMEM((WE, 128), jnp.float32),
            pltpu.VMEM((WE, 128), jnp.float32),
            pltpu.VMEM((2 * WE,), jnp.int32),
            pltpu.VMEM((2 * WE,), jnp.int32),
            pltpu.VMEM((WE,), jnp.int32),
            pltpu.VMEM((WE,), jnp.int32),
            pltpu.VMEM((WE,), jnp.int32),
            pltpu.VMEM((WE,), jnp.int32),
            pltpu.VMEM((WE,), jnp.float32),
            pltpu.VMEM((WE,), jnp.float32),
            pltpu.VMEM((WE,), jnp.float32),
            pltpu.VMEM((RPT,), jnp.float32),
            pltpu.SemaphoreType.DMA,
            pltpu.SemaphoreType.DMA,
            pltpu.SemaphoreType.DMA,
            pltpu.SemaphoreType.DMA,
            pltpu.VMEM_SHARED((NA, 128), jnp.float32),
            pltpu.VMEM_SHARED((NA,), jnp.float32),
        ],
        compiler_params=_sc_compiler_params(),
    )
    return run(*h_chunks, sdp_flat, ss_flat, ds_packed)


def _gat_layer(x, ds_packed, W, a, merge):
    H = W.shape[0]
    h_chunks, sd, ss = _dense_heads(x, W, a)
    nchunks = len(h_chunks)
    sdp = jnp.pad(sd, ((0, 0), (0, NA - N))).reshape(-1)
    ssf = ss.reshape(-1)
    flat = [hh.reshape(H * N, 128) for hh in h_chunks]
    outs = {}
    for p in range(H // 2):
        o = _sc_pair(flat, sdp, ssf, ds_packed, 2 * p, merge == "concat")
        o = o.reshape(2, nchunks, NA, 128)
        for cc in range(2):
            for ci in range(nchunks):
                outs[(2 * p + cc, ci)] = o[cc, ci, :N]
    if merge == "concat":
        cols = [outs[(hd, ci)] for hd in range(H) for ci in range(nchunks)]
        return jnp.concatenate(cols, axis=1)
    acc = outs[(0, 0)]
    for hd in range(1, H):
        acc = acc + outs[(hd, 0)]
    return jax.nn.relu(acc / H)


def kernel(node_states, edges, training, W1, a1, W2, a2, W3, a3):
    dst = edges[:, 0]
    src = edges[:, 1]
    pad = E_SC - E
    e = jnp.arange(pad, dtype=jnp.int32)
    dst_sc = jnp.concatenate([dst, N + (e % 64)])
    src_sc = jnp.concatenate([src, e % N])
    ds_packed = jnp.concatenate(
        [dst_sc.reshape(-1, WE), src_sc.reshape(-1, WE)], axis=1).reshape(-1)
    x = _gat_layer(node_states, ds_packed, W1, a1, "concat")
    x = _gat_layer(x, ds_packed, W2, a2, "concat")
    return _gat_layer(x, ds_packed, W3, a3, "avg")


# speedup vs baseline: 1.4625x; 1.0068x over previous
import dataclasses
import functools

import jax
import jax.numpy as jnp
from jax import lax
from jax.experimental import pallas as pl
from jax.experimental.pallas import tpu as pltpu
from jax.experimental.pallas import tpu_sc as plsc

N = 10000
E = 160000
NA = 10240
RPT = NA // 16
WE = 64
WPT = 160
EPT = WPT * WE
E_SC = 16 * EPT


def _dense_kernel(nchunks, x_ref, w_ref, ast_ref, *out_refs):
    x = x_ref[...]
    w = w_ref[0]
    h = jax.lax.dot_general(x, w, (((1,), (0,)), ((), ())),
                            preferred_element_type=jnp.float32)
    for k in range(nchunks):
        out_refs[k][0] = h[:, k * 128:(k + 1) * 128]
    out_refs[nchunks][0] = jax.lax.dot_general(
        h, ast_ref[0], (((1,), (1,)), ((), ())),
        preferred_element_type=jnp.float32)


def _dense_heads(x, W, a, bn=400):
    H, Din, u = W.shape
    nchunks = u // 128
    nb = N // bn
    a2 = a[:, :, 0].reshape(H, 2, u)
    ast = jnp.concatenate([a2, jnp.zeros((H, 6, u), jnp.float32)], axis=1)
    outs = pl.pallas_call(
        functools.partial(_dense_kernel, nchunks),
        grid=(H, nb),
        in_specs=[
            pl.BlockSpec((bn, Din), lambda i, n: (n, 0)),
            pl.BlockSpec((1, Din, u), lambda i, n: (i, 0, 0)),
            pl.BlockSpec((1, 8, u), lambda i, n: (i, 0, 0)),
        ],
        out_specs=[pl.BlockSpec((1, bn, 128), lambda i, n: (i, n, 0))
                   for _ in range(nchunks)]
                  + [pl.BlockSpec((1, bn, 8), lambda i, n: (i, n, 0))],
        out_shape=[jax.ShapeDtypeStruct((H, N, 128), jnp.float32)
                   for _ in range(nchunks)]
                  + [jax.ShapeDtypeStruct((H, N, 8), jnp.float32)],
    )(x, W, ast)
    h_chunks, scal = outs[:nchunks], outs[nchunks]
    return h_chunks, scal[:, :, 0], scal[:, :, 1]


def _sc_gat_body(relu, pair_base, nchunks, *refs):
    h_hbms = refs[:nchunks]
    sdp_hbm, ss_hbm, ds_hbm, out_hbm = refs[nchunks:nchunks + 4]
    (sdp_loc, ss_loc, rows0, rows1, dsb0, dsb1, dstw0, dstw1,
     gidx0, gidx1, s_all, dloc, zvec,
     g_sem0, g_sem1, t_sem0, t_sem1,
     acc_sh, den_sh) = refs[nchunks + 4:]
    rows = (rows0, rows1)
    dsb = (dsb0, dsb1)
    dstw = (dstw0, dstw1)
    gidx = (gidx0, gidx1)
    g_sem = (g_sem0, g_sem1)
    t_sem = (t_sem0, t_sem1)

    c = lax.axis_index("c")
    s = lax.axis_index("s")
    head = pair_base + c
    zero16 = jnp.zeros((16,), jnp.float32)

    pltpu.sync_copy(sdp_hbm.at[pl.ds(head * NA, NA)], sdp_loc)
    pltpu.sync_copy(ss_hbm.at[pl.ds(head * N, N)], ss_loc)

    def zero_rows(r_ref):
        @pl.loop(0, WE)
        def _z(r):
            for j in range(8):
                r_ref[r, pl.ds(j * 16, 16)] = zero16

    def zero_acc():
        for k in range(RPT // WE):
            pltpu.sync_copy(rows0, acc_sh.at[pl.ds(s * RPT + k * WE, WE)])

    zero_rows(rows0)

    @pl.loop(0, RPT // 16)
    def _zv(i):
        zvec[pl.ds(i * 16, 16)] = zero16

    zero_acc()
    pltpu.sync_copy(zvec, den_sh.at[pl.ds(s * RPT, RPT)])
    plsc.subcore_barrier()

    def load_idx(w, b):
        pltpu.sync_copy(
            ds_hbm.at[pl.ds((s * WPT + w) * 2 * WE, 2 * WE)], dsb[b])

    def scores(w, b):
        for j in range(WE // 16):
            di = dsb[b][pl.ds(j * 16, 16)]
            si = dsb[b][pl.ds(WE + j * 16, 16)]
            t = plsc.load_gather(sdp_loc, [di]) + plsc.load_gather(ss_loc, [si])
            t = jnp.where(t > 0, t, 0.2 * t)
            t = jnp.minimum(jnp.maximum(t, -2.0), 2.0)
            s_all[pl.ds(w * WE + j * 16, 16)] = jnp.exp(t)
            gidx[b][pl.ds(j * 16, 16)] = si + head * N
            dstw[b][pl.ds(j * 16, 16)] = di

    def reindex(b):
        for j in range(WE // 16):
            di = dsb[b][pl.ds(j * 16, 16)]
            si = dsb[b][pl.ds(WE + j * 16, 16)]
            gidx[b][pl.ds(j * 16, 16)] = si + head * N
            dstw[b][pl.ds(j * 16, 16)] = di

    def gather(b, h_hbm):
        return pltpu.make_async_copy(h_hbm.at[gidx[b]], rows[b], g_sem[b])

    def scatter(b):
        return pltpu.make_async_copy(rows[b], acc_sh.at[dstw[b]], t_sem[b])

    def scale(w, b):
        @pl.loop(0, WE)
        def _scale(r):
            sv = plsc.load_gather(s_all, [jnp.full((16,), w * WE + r,
                                                   jnp.int32)])
            for j in range(8):
                rows[b][r, pl.ds(j * 16, 16)] = (
                    rows[b][r, pl.ds(j * 16, 16)] * sv)

    def flush(chunk):
        for k in range(RPT // WE):
            row0 = s * RPT + k * WE
            pltpu.sync_copy(acc_sh.at[pl.ds(row0, WE)], rows0)
            pltpu.sync_copy(den_sh.at[pl.ds(row0, WE)], dloc)

            @pl.loop(0, WE)
            def _norm(r):
                dv = plsc.load_gather(dloc, [jnp.full((16,), r, jnp.int32)])
                sc = 1.0 / (dv + 1e-9)
                for j in range(8):
                    v = rows0[r, pl.ds(j * 16, 16)] * sc
                    if relu:
                        v = jnp.maximum(v, 0.0)
                    rows0[r, pl.ds(j * 16, 16)] = v

            pltpu.sync_copy(
                rows0, out_hbm.at[pl.ds((c * nchunks + chunk) * NA + row0,
                                        WE)])

    def edge_pass(chunk, h_hbm):
        first = chunk == 0

        def prep(w, b):
            load_idx(w, b)
            if first:
                scores(w, b)
            else:
                reindex(b)

        def body(w, b, prefetch=True):
            nb = 1 - b
            if prefetch:
                prep(w + 1, nb)
                gather(nb, h_hbm).start()
            gather(b, h_hbm).wait()
            if first:
                pltpu.sync_copy(s_all.at[pl.ds(w * WE, WE)],
                                den_sh.at[dstw[b]], add=True)
            scale(w, b)
            pltpu.sync_copy(rows[b], acc_sh.at[dstw[b]], add=True)

        prep(0, 0)
        gather(0, h_hbm).start()
        body(0, 0)

        @pl.loop(1, WPT - 3, step=2)
        def _pair(w0):
            body(w0, 1)
            body(w0 + 1, 0)

        body(WPT - 3, 1)
        body(WPT - 2, 0)
        body(WPT - 1, 1, prefetch=False)
        plsc.subcore_barrier()
        flush(chunk)
        if chunk + 1 < nchunks:
            plsc.subcore_barrier()
            zero_rows(rows0)
            zero_acc()
            plsc.subcore_barrier()

    for chunk in range(nchunks):
        edge_pass(chunk, h_hbms[chunk])


def _sc_compiler_params():
    cp = pltpu.CompilerParams()
    if "needs_layout_passes" in pltpu.CompilerParams.__dataclass_fields__:
        cp = dataclasses.replace(cp, needs_layout_passes=False)
    return cp


def _sc_pair(h_chunks, sdp_flat, ss_flat, ds_packed, pair_base, relu):
    nchunks = len(h_chunks)
    mesh = plsc.VectorSubcoreMesh(core_axis_name="c", subcore_axis_name="s")
    body = functools.partial(_sc_gat_body, relu, pair_base, nchunks)
    run = pl.kernel(
        body,
        out_type=jax.ShapeDtypeStruct((2 * nchunks * NA, 128), jnp.float32),
        mesh=mesh,
        scratch_types=[
            pltpu.VMEM((NA,), jnp.float32),
            pltpu.VMEM((N,), jnp.float32),
            pltpu.VMEM((WE, 128), jnp.float32),
            pltpu.VMEM((WE, 128), jnp.float32),
            pltpu.VMEM((2 * WE,), jnp.int32),
            pltpu.VMEM((2 * WE,), jnp.int32),
            pltpu.VMEM((WE,), jnp.int32),
            pltpu.VMEM((WE,), jnp.int32),
            pltpu.VMEM((WE,), jnp.int32),
            pltpu.VMEM((WE,), jnp.int32),
            pltpu.VMEM((EPT,), jnp.float32),
            pltpu.VMEM((WE,), jnp.float32),
            pltpu.VMEM((RPT,), jnp.float32),
            pltpu.SemaphoreType.DMA,
            pltpu.SemaphoreType.DMA,
            pltpu.SemaphoreType.DMA,
            pltpu.SemaphoreType.DMA,
            pltpu.VMEM_SHARED((NA, 128), jnp.float32),
            pltpu.VMEM_SHARED((NA,), jnp.float32),
        ],
        compiler_params=_sc_compiler_params(),
    )
    return run(*h_chunks, sdp_flat, ss_flat, ds_packed)


def _gat_layer(x, ds_packed, W, a, merge):
    H = W.shape[0]
    h_chunks, sd, ss = _dense_heads(x, W, a)
    nchunks = len(h_chunks)
    sdp = jnp.pad(sd, ((0, 0), (0, NA - N))).reshape(-1)
    ssf = ss.reshape(-1)
    flat = [hh.reshape(H * N, 128) for hh in h_chunks]
    outs = {}
    for p in range(H // 2):
        o = _sc_pair(flat, sdp, ssf, ds_packed, 2 * p, merge == "concat")
        o = o.reshape(2, nchunks, NA, 128)
        for cc in range(2):
            for ci in range(nchunks):
                outs[(2 * p + cc, ci)] = o[cc, ci, :N]
    if merge == "concat":
        cols = [outs[(hd, ci)] for hd in range(H) for ci in range(nchunks)]
        return jnp.concatenate(cols, axis=1)
    acc = outs[(0, 0)]
    for hd in range(1, H):
        acc = acc + outs[(hd, 0)]
    return jax.nn.relu(acc / H)


def kernel(node_states, edges, training, W1, a1, W2, a2, W3, a3):
    dst = edges[:, 0]
    src = edges[:, 1]
    pad = E_SC - E
    e = jnp.arange(pad, dtype=jnp.int32)
    dst_sc = jnp.concatenate([dst, N + (e % 64)])
    src_sc = jnp.concatenate([src, e % N])
    ds_packed = jnp.concatenate(
        [dst_sc.reshape(-1, WE), src_sc.reshape(-1, WE)], axis=1).reshape(-1)
    x = _gat_layer(node_states, ds_packed, W1, a1, "concat")
    x = _gat_layer(x, ds_packed, W2, a2, "concat")
    return _gat_layer(x, ds_packed, W3, a3, "avg")


# speedup vs baseline: 1.7245x; 1.1792x over previous
import dataclasses
import functools

import jax
import jax.numpy as jnp
from jax import lax
from jax.experimental import pallas as pl
from jax.experimental.pallas import tpu as pltpu
from jax.experimental.pallas import tpu_sc as plsc

N = 10000
E = 160000
NA = 10240
RPT = NA // 16
WE = 64
WPT = 160
EPT = WPT * WE
E_SC = 16 * EPT


def _dense_kernel(nchunks, x_ref, w_ref, ast_ref, *out_refs):
    x = x_ref[...]
    w = w_ref[0]
    h = jax.lax.dot_general(x, w, (((1,), (0,)), ((), ())),
                            preferred_element_type=jnp.float32)
    for k in range(nchunks):
        out_refs[k][0] = h[:, k * 128:(k + 1) * 128]
    out_refs[nchunks][0] = jax.lax.dot_general(
        h, ast_ref[0], (((1,), (1,)), ((), ())),
        preferred_element_type=jnp.float32)


def _dense_heads(x, W, a, bn=400):
    H, Din, u = W.shape
    nchunks = u // 128
    nb = N // bn
    a2 = a[:, :, 0].reshape(H, 2, u)
    ast = jnp.concatenate([a2, jnp.zeros((H, 6, u), jnp.float32)], axis=1)
    outs = pl.pallas_call(
        functools.partial(_dense_kernel, nchunks),
        grid=(H, nb),
        in_specs=[
            pl.BlockSpec((bn, Din), lambda i, n: (n, 0)),
            pl.BlockSpec((1, Din, u), lambda i, n: (i, 0, 0)),
            pl.BlockSpec((1, 8, u), lambda i, n: (i, 0, 0)),
        ],
        out_specs=[pl.BlockSpec((1, bn, 128), lambda i, n: (i, n, 0))
                   for _ in range(nchunks)]
                  + [pl.BlockSpec((1, bn, 8), lambda i, n: (i, n, 0))],
        out_shape=[jax.ShapeDtypeStruct((H, N, 128), jnp.float32)
                   for _ in range(nchunks)]
                  + [jax.ShapeDtypeStruct((H, N, 8), jnp.float32)],
    )(x, W, ast)
    h_chunks, scal = outs[:nchunks], outs[nchunks]
    return h_chunks, scal[:, :, 0], scal[:, :, 1]


def _sc_gat_body(relu, pair_base, nchunks, *refs):
    h_hbms = refs[:nchunks]
    sdp_hbm, ss_hbm, ds_hbm, out_hbm = refs[nchunks:nchunks + 4]
    (sdp_loc, ss_loc, rows0, rows1, dsb0, dsb1, dstw0, dstw1,
     gidx0, gidx1, s_all, dloc, zvec,
     g_sem0, g_sem1, t_sem0, t_sem1,
     acc_sh, den_sh) = refs[nchunks + 4:]
    rows = (rows0, rows1)
    dsb = (dsb0, dsb1)
    dstw = (dstw0, dstw1)
    gidx = (gidx0, gidx1)
    g_sem = (g_sem0, g_sem1)
    t_sem = (t_sem0, t_sem1)

    c = lax.axis_index("c")
    s = lax.axis_index("s")
    head = pair_base + c
    zero16 = jnp.zeros((16,), jnp.float32)

    pltpu.sync_copy(sdp_hbm.at[pl.ds(head * NA, NA)], sdp_loc)
    pltpu.sync_copy(ss_hbm.at[pl.ds(head * N, N)], ss_loc)

    def zero_rows(r_ref):
        @pl.loop(0, WE)
        def _z(r):
            for j in range(8):
                r_ref[r, pl.ds(j * 16, 16)] = zero16

    def zero_acc():
        for k in range(RPT // WE):
            pltpu.sync_copy(rows0, acc_sh.at[pl.ds(s * RPT + k * WE, WE)])

    zero_rows(rows0)

    @pl.loop(0, RPT // 16)
    def _zv(i):
        zvec[pl.ds(i * 16, 16)] = zero16

    zero_acc()
    pltpu.sync_copy(zvec, den_sh.at[pl.ds(s * RPT, RPT)])
    plsc.subcore_barrier()

    def load_idx(w, b):
        pltpu.sync_copy(
            ds_hbm.at[pl.ds((s * WPT + w) * 2 * WE, 2 * WE)], dsb[b])

    def scores(w, b):
        for j in range(WE // 16):
            di = dsb[b][pl.ds(j * 16, 16)]
            si = dsb[b][pl.ds(WE + j * 16, 16)]
            t = plsc.load_gather(sdp_loc, [di]) + plsc.load_gather(ss_loc, [si])
            t = jnp.where(t > 0, t, 0.2 * t)
            t = jnp.minimum(jnp.maximum(t, -2.0), 2.0)
            s_all[pl.ds(w * WE + j * 16, 16)] = jnp.exp(t)
            gidx[b][pl.ds(j * 16, 16)] = si + head * N
            dstw[b][pl.ds(j * 16, 16)] = di

    def reindex(b):
        for j in range(WE // 16):
            di = dsb[b][pl.ds(j * 16, 16)]
            si = dsb[b][pl.ds(WE + j * 16, 16)]
            gidx[b][pl.ds(j * 16, 16)] = si + head * N
            dstw[b][pl.ds(j * 16, 16)] = di

    def gather(b, h_hbm):
        return pltpu.make_async_copy(h_hbm.at[gidx[b]], rows[b], g_sem[b])

    def scatter(b):
        return pltpu.make_async_copy(rows[b], acc_sh.at[dstw[b]], t_sem[b])

    def scale(w, b):
        @plsc.parallel_loop(0, WE, unroll=4)
        def _scale(r):
            sv = plsc.load_gather(s_all, [jnp.full((16,), w * WE + r,
                                                   jnp.int32)])
            for j in range(8):
                rows[b][r, pl.ds(j * 16, 16)] = (
                    rows[b][r, pl.ds(j * 16, 16)] * sv)

    def flush(chunk):
        for k in range(RPT // WE):
            row0 = s * RPT + k * WE
            pltpu.sync_copy(acc_sh.at[pl.ds(row0, WE)], rows0)
            pltpu.sync_copy(den_sh.at[pl.ds(row0, WE)], dloc)

            @pl.loop(0, WE)
            def _norm(r):
                dv = plsc.load_gather(dloc, [jnp.full((16,), r, jnp.int32)])
                sc = 1.0 / (dv + 1e-9)
                for j in range(8):
                    v = rows0[r, pl.ds(j * 16, 16)] * sc
                    if relu:
                        v = jnp.maximum(v, 0.0)
                    rows0[r, pl.ds(j * 16, 16)] = v

            pltpu.sync_copy(
                rows0, out_hbm.at[pl.ds((c * nchunks + chunk) * NA + row0,
                                        WE)])

    def edge_pass(chunk, h_hbm):
        first = chunk == 0

        def prep(w, b):
            load_idx(w, b)
            if first:
                scores(w, b)
            else:
                reindex(b)

        def body(w, b, prefetch=True):
            nb = 1 - b
            if prefetch:
                prep(w + 1, nb)
                gather(nb, h_hbm).start()
            gather(b, h_hbm).wait()
            if first:
                pltpu.sync_copy(s_all.at[pl.ds(w * WE, WE)],
                                den_sh.at[dstw[b]], add=True)
            scale(w, b)
            pltpu.sync_copy(rows[b], acc_sh.at[dstw[b]], add=True)

        prep(0, 0)
        gather(0, h_hbm).start()
        body(0, 0)

        @pl.loop(1, WPT - 3, step=2)
        def _pair(w0):
            body(w0, 1)
            body(w0 + 1, 0)

        body(WPT - 3, 1)
        body(WPT - 2, 0)
        body(WPT - 1, 1, prefetch=False)
        plsc.subcore_barrier()
        flush(chunk)
        if chunk + 1 < nchunks:
            plsc.subcore_barrier()
            zero_rows(rows0)
            zero_acc()
            plsc.subcore_barrier()

    for chunk in range(nchunks):
        edge_pass(chunk, h_hbms[chunk])


def _sc_compiler_params():
    cp = pltpu.CompilerParams()
    if "needs_layout_passes" in pltpu.CompilerParams.__dataclass_fields__:
        cp = dataclasses.replace(cp, needs_layout_passes=False)
    return cp


def _sc_pair(h_chunks, sdp_flat, ss_flat, ds_packed, pair_base, relu):
    nchunks = len(h_chunks)
    mesh = plsc.VectorSubcoreMesh(core_axis_name="c", subcore_axis_name="s")
    body = functools.partial(_sc_gat_body, relu, pair_base, nchunks)
    run = pl.kernel(
        body,
        out_type=jax.ShapeDtypeStruct((2 * nchunks * NA, 128), jnp.float32),
        mesh=mesh,
        scratch_types=[
            pltpu.VMEM((NA,), jnp.float32),
            pltpu.VMEM((N,), jnp.float32),
            pltpu.VMEM((WE, 128), jnp.float32),
            pltpu.VMEM((WE, 128), jnp.float32),
            pltpu.VMEM((2 * WE,), jnp.int32),
            pltpu.VMEM((2 * WE,), jnp.int32),
            pltpu.VMEM((WE,), jnp.int32),
            pltpu.VMEM((WE,), jnp.int32),
            pltpu.VMEM((WE,), jnp.int32),
            pltpu.VMEM((WE,), jnp.int32),
            pltpu.VMEM((EPT,), jnp.float32),
            pltpu.VMEM((WE,), jnp.float32),
            pltpu.VMEM((RPT,), jnp.float32),
            pltpu.SemaphoreType.DMA,
            pltpu.SemaphoreType.DMA,
            pltpu.SemaphoreType.DMA,
            pltpu.SemaphoreType.DMA,
            pltpu.VMEM_SHARED((NA, 128), jnp.float32),
            pltpu.VMEM_SHARED((NA,), jnp.float32),
        ],
        compiler_params=_sc_compiler_params(),
    )
    return run(*h_chunks, sdp_flat, ss_flat, ds_packed)


def _gat_layer(x, ds_packed, W, a, merge):
    H = W.shape[0]
    h_chunks, sd, ss = _dense_heads(x, W, a)
    nchunks = len(h_chunks)
    sdp = jnp.pad(sd, ((0, 0), (0, NA - N))).reshape(-1)
    ssf = ss.reshape(-1)
    flat = [hh.reshape(H * N, 128) for hh in h_chunks]
    outs = {}
    for p in range(H // 2):
        o = _sc_pair(flat, sdp, ssf, ds_packed, 2 * p, merge == "concat")
        o = o.reshape(2, nchunks, NA, 128)
        for cc in range(2):
            for ci in range(nchunks):
                outs[(2 * p + cc, ci)] = o[cc, ci, :N]
    if merge == "concat":
        cols = [outs[(hd, ci)] for hd in range(H) for ci in range(nchunks)]
        return jnp.concatenate(cols, axis=1)
    acc = outs[(0, 0)]
    for hd in range(1, H):
        acc = acc + outs[(hd, 0)]
    return jax.nn.relu(acc / H)


def kernel(node_states, edges, training, W1, a1, W2, a2, W3, a3):
    dst = edges[:, 0]
    src = edges[:, 1]
    pad = E_SC - E
    e = jnp.arange(pad, dtype=jnp.int32)
    dst_sc = jnp.concatenate([dst, N + (e % 64)])
    src_sc = jnp.concatenate([src, e % N])
    ds_packed = jnp.concatenate(
        [dst_sc.reshape(-1, WE), src_sc.reshape(-1, WE)], axis=1).reshape(-1)
    x = _gat_layer(node_states, ds_packed, W1, a1, "concat")
    x = _gat_layer(x, ds_packed, W2, a2, "concat")
    return _gat_layer(x, ds_packed, W3, a3, "avg")


# speedup vs baseline: 1.7622x; 1.0219x over previous
import dataclasses
import functools

import jax
import jax.numpy as jnp
from jax import lax
from jax.experimental import pallas as pl
from jax.experimental.pallas import tpu as pltpu
from jax.experimental.pallas import tpu_sc as plsc

N = 10000
E = 160000
NA = 10240
RPT = NA // 16
WE = 64
WPT = 160
EPT = WPT * WE
E_SC = 16 * EPT


def _dense_kernel(nchunks, x_ref, w_ref, ast_ref, *out_refs):
    x = x_ref[...]
    w = w_ref[0]
    h = jax.lax.dot_general(x, w, (((1,), (0,)), ((), ())),
                            preferred_element_type=jnp.float32)
    for k in range(nchunks):
        out_refs[k][0] = h[:, k * 128:(k + 1) * 128]
    out_refs[nchunks][0] = jax.lax.dot_general(
        h, ast_ref[0], (((1,), (1,)), ((), ())),
        preferred_element_type=jnp.float32)


def _dense_heads(x, W, a, bn=400):
    H, Din, u = W.shape
    nchunks = u // 128
    nb = N // bn
    a2 = a[:, :, 0].reshape(H, 2, u)
    ast = jnp.concatenate([a2, jnp.zeros((H, 6, u), jnp.float32)], axis=1)
    outs = pl.pallas_call(
        functools.partial(_dense_kernel, nchunks),
        grid=(H, nb),
        in_specs=[
            pl.BlockSpec((bn, Din), lambda i, n: (n, 0)),
            pl.BlockSpec((1, Din, u), lambda i, n: (i, 0, 0)),
            pl.BlockSpec((1, 8, u), lambda i, n: (i, 0, 0)),
        ],
        out_specs=[pl.BlockSpec((1, bn, 128), lambda i, n: (i, n, 0))
                   for _ in range(nchunks)]
                  + [pl.BlockSpec((1, bn, 8), lambda i, n: (i, n, 0))],
        out_shape=[jax.ShapeDtypeStruct((H, N, 128), jnp.float32)
                   for _ in range(nchunks)]
                  + [jax.ShapeDtypeStruct((H, N, 8), jnp.float32)],
    )(x, W, ast)
    h_chunks, scal = outs[:nchunks], outs[nchunks]
    return h_chunks, scal[:, :, 0], scal[:, :, 1]


def _sc_gat_body(relu, pair_base, nchunks, *refs):
    h_hbms = refs[:nchunks]
    sdp_hbm, ss_hbm, ds_hbm, out_hbm = refs[nchunks:nchunks + 4]
    (sdp_loc, ss_loc, rows0, rows1, dsb0, dsb1, dstw0, dstw1,
     gidx0, gidx1, s_all, dloc, zvec,
     g_sem0, g_sem1, t_sem0, t_sem1,
     acc_sh, den_sh) = refs[nchunks + 4:]
    rows = (rows0, rows1)
    dsb = (dsb0, dsb1)
    dstw = (dstw0, dstw1)
    gidx = (gidx0, gidx1)
    g_sem = (g_sem0, g_sem1)
    t_sem = (t_sem0, t_sem1)

    c = lax.axis_index("c")
    s = lax.axis_index("s")
    head = pair_base + c
    zero16 = jnp.zeros((16,), jnp.float32)

    pltpu.sync_copy(sdp_hbm.at[pl.ds(head * NA, NA)], sdp_loc)
    pltpu.sync_copy(ss_hbm.at[pl.ds(head * N, N)], ss_loc)

    def zero_rows(r_ref):
        @plsc.parallel_loop(0, WE, unroll=4)
        def _z(r):
            for j in range(8):
                r_ref[r, pl.ds(j * 16, 16)] = zero16

    def zero_acc():
        for k in range(RPT // WE):
            pltpu.sync_copy(rows0, acc_sh.at[pl.ds(s * RPT + k * WE, WE)])

    zero_rows(rows0)

    @pl.loop(0, RPT // 16)
    def _zv(i):
        zvec[pl.ds(i * 16, 16)] = zero16

    zero_acc()
    pltpu.sync_copy(zvec, den_sh.at[pl.ds(s * RPT, RPT)])
    plsc.subcore_barrier()

    def load_idx(w, b):
        pltpu.sync_copy(
            ds_hbm.at[pl.ds((s * WPT + w) * 2 * WE, 2 * WE)], dsb[b])

    def scores(w, b):
        for j in range(WE // 16):
            di = dsb[b][pl.ds(j * 16, 16)]
            si = dsb[b][pl.ds(WE + j * 16, 16)]
            t = plsc.load_gather(sdp_loc, [di]) + plsc.load_gather(ss_loc, [si])
            t = jnp.where(t > 0, t, 0.2 * t)
            t = jnp.minimum(jnp.maximum(t, -2.0), 2.0)
            s_all[pl.ds(w * WE + j * 16, 16)] = jnp.exp(t)
            gidx[b][pl.ds(j * 16, 16)] = si + head * N
            dstw[b][pl.ds(j * 16, 16)] = di

    def reindex(b):
        for j in range(WE // 16):
            di = dsb[b][pl.ds(j * 16, 16)]
            si = dsb[b][pl.ds(WE + j * 16, 16)]
            gidx[b][pl.ds(j * 16, 16)] = si + head * N
            dstw[b][pl.ds(j * 16, 16)] = di

    def gather(b, h_hbm):
        return pltpu.make_async_copy(h_hbm.at[gidx[b]], rows[b], g_sem[b])

    def scatter(b):
        return pltpu.make_async_copy(rows[b], acc_sh.at[dstw[b]], t_sem[b])

    def scale(w, b):
        @plsc.parallel_loop(0, WE, unroll=4)
        def _scale(r):
            sv = plsc.load_gather(s_all, [jnp.full((16,), w * WE + r,
                                                   jnp.int32)])
            for j in range(8):
                rows[b][r, pl.ds(j * 16, 16)] = (
                    rows[b][r, pl.ds(j * 16, 16)] * sv)

    def flush(chunk):
        for k in range(RPT // WE):
            row0 = s * RPT + k * WE
            pltpu.sync_copy(acc_sh.at[pl.ds(row0, WE)], rows0)
            pltpu.sync_copy(den_sh.at[pl.ds(row0, WE)], dloc)

            @plsc.parallel_loop(0, WE, unroll=4)
            def _norm(r):
                dv = plsc.load_gather(dloc, [jnp.full((16,), r, jnp.int32)])
                sc = 1.0 / (dv + 1e-9)
                for j in range(8):
                    v = rows0[r, pl.ds(j * 16, 16)] * sc
                    if relu:
                        v = jnp.maximum(v, 0.0)
                    rows0[r, pl.ds(j * 16, 16)] = v

            pltpu.sync_copy(
                rows0, out_hbm.at[pl.ds((c * nchunks + chunk) * NA + row0,
                                        WE)])

    def edge_pass(chunk, h_hbm):
        first = chunk == 0

        def prep(w, b):
            load_idx(w, b)
            if first:
                scores(w, b)
            else:
                reindex(b)

        def body(w, b, prefetch=True):
            nb = 1 - b
            if prefetch:
                prep(w + 1, nb)
                gather(nb, h_hbm).start()
            gather(b, h_hbm).wait()
            if first:
                pltpu.sync_copy(s_all.at[pl.ds(w * WE, WE)],
                                den_sh.at[dstw[b]], add=True)
            scale(w, b)
            pltpu.sync_copy(rows[b], acc_sh.at[dstw[b]], add=True)

        prep(0, 0)
        gather(0, h_hbm).start()
        body(0, 0)

        @pl.loop(1, WPT - 3, step=2)
        def _pair(w0):
            body(w0, 1)
            body(w0 + 1, 0)

        body(WPT - 3, 1)
        body(WPT - 2, 0)
        body(WPT - 1, 1, prefetch=False)
        plsc.subcore_barrier()
        flush(chunk)
        if chunk + 1 < nchunks:
            plsc.subcore_barrier()
            zero_rows(rows0)
            zero_acc()
            plsc.subcore_barrier()

    for chunk in range(nchunks):
        edge_pass(chunk, h_hbms[chunk])


def _sc_compiler_params():
    cp = pltpu.CompilerParams()
    if "needs_layout_passes" in pltpu.CompilerParams.__dataclass_fields__:
        cp = dataclasses.replace(cp, needs_layout_passes=False)
    return cp


def _sc_pair(h_chunks, sdp_flat, ss_flat, ds_packed, pair_base, relu):
    nchunks = len(h_chunks)
    mesh = plsc.VectorSubcoreMesh(core_axis_name="c", subcore_axis_name="s")
    body = functools.partial(_sc_gat_body, relu, pair_base, nchunks)
    run = pl.kernel(
        body,
        out_type=jax.ShapeDtypeStruct((2 * nchunks * NA, 128), jnp.float32),
        mesh=mesh,
        scratch_types=[
            pltpu.VMEM((NA,), jnp.float32),
            pltpu.VMEM((N,), jnp.float32),
            pltpu.VMEM((WE, 128), jnp.float32),
            pltpu.VMEM((WE, 128), jnp.float32),
            pltpu.VMEM((2 * WE,), jnp.int32),
            pltpu.VMEM((2 * WE,), jnp.int32),
            pltpu.VMEM((WE,), jnp.int32),
            pltpu.VMEM((WE,), jnp.int32),
            pltpu.VMEM((WE,), jnp.int32),
            pltpu.VMEM((WE,), jnp.int32),
            pltpu.VMEM((EPT,), jnp.float32),
            pltpu.VMEM((WE,), jnp.float32),
            pltpu.VMEM((RPT,), jnp.float32),
            pltpu.SemaphoreType.DMA,
            pltpu.SemaphoreType.DMA,
            pltpu.SemaphoreType.DMA,
            pltpu.SemaphoreType.DMA,
            pltpu.VMEM_SHARED((NA, 128), jnp.float32),
            pltpu.VMEM_SHARED((NA,), jnp.float32),
        ],
        compiler_params=_sc_compiler_params(),
    )
    return run(*h_chunks, sdp_flat, ss_flat, ds_packed)


def _gat_layer(x, ds_packed, W, a, merge):
    H = W.shape[0]
    h_chunks, sd, ss = _dense_heads(x, W, a)
    nchunks = len(h_chunks)
    sdp = jnp.pad(sd, ((0, 0), (0, NA - N))).reshape(-1)
    ssf = ss.reshape(-1)
    flat = [hh.reshape(H * N, 128) for hh in h_chunks]
    outs = {}
    for p in range(H // 2):
        o = _sc_pair(flat, sdp, ssf, ds_packed, 2 * p, merge == "concat")
        o = o.reshape(2, nchunks, NA, 128)
        for cc in range(2):
            for ci in range(nchunks):
                outs[(2 * p + cc, ci)] = o[cc, ci, :N]
    if merge == "concat":
        cols = [outs[(hd, ci)] for hd in range(H) for ci in range(nchunks)]
        return jnp.concatenate(cols, axis=1)
    acc = outs[(0, 0)]
    for hd in range(1, H):
        acc = acc + outs[(hd, 0)]
    return jax.nn.relu(acc / H)


def kernel(node_states, edges, training, W1, a1, W2, a2, W3, a3):
    dst = edges[:, 0]
    src = edges[:, 1]
    pad = E_SC - E
    e = jnp.arange(pad, dtype=jnp.int32)
    dst_sc = jnp.concatenate([dst, N + (e % 64)])
    src_sc = jnp.concatenate([src, e % N])
    ds_packed = jnp.concatenate(
        [dst_sc.reshape(-1, WE), src_sc.reshape(-1, WE)], axis=1).reshape(-1)
    x = _gat_layer(node_states, ds_packed, W1, a1, "concat")
    x = _gat_layer(x, ds_packed, W2, a2, "concat")
    return _gat_layer(x, ds_packed, W3, a3, "avg")
